# baseline (speedup 1.0000x reference)
import jax
import jax.numpy as jnp
from jax.experimental import pallas as pl

_N = 8192
_D = 64
_CUTOFF = 0.5
_MU = 1.0
_SIGMA = 0.5
_INV2SIG2 = 1.0 / (2.0 * _SIGMA * _SIGMA)

_BM = 512
_BK = 2048


def _msg_kernel(z_ref, w_ref, b_ref, out_ref):
    out_ref[...] = (
        jnp.dot(z_ref[...], w_ref[...], preferred_element_type=jnp.float32)
        + b_ref[...]
    ).astype(jnp.bfloat16)


def _interact_kernel(dist_ref, msg_ref, out_ref):
    k = pl.program_id(1)
    r = dist_ref[...]
    u = 1.0 / r
    expo = -_INV2SIG2 * (u - 1.0 / _MU) ** 2
    w = jnp.where(r < _CUTOFF, jnp.exp(expo), 0.0).astype(jnp.bfloat16)
    msg_blk = msg_ref[pl.ds(k * _BK, _BK), :]
    part = jnp.dot(w, msg_blk, preferred_element_type=jnp.float32)

    @pl.when(k == 0)
    def _init():
        out_ref[...] = part

    @pl.when(k != 0)
    def _acc():
        out_ref[...] += part


def kernel(z, dist_matrix, W, B):
    msg = pl.pallas_call(
        _msg_kernel,
        out_shape=jax.ShapeDtypeStruct((_N, _D), jnp.bfloat16),
    )(z, W, B.reshape(1, _D))

    out = pl.pallas_call(
        _interact_kernel,
        grid=(_N // _BM, _N // _BK),
        in_specs=[
            pl.BlockSpec((_BM, _BK), lambda i, k: (i, k)),
            pl.BlockSpec((_N, _D), lambda i, k: (0, 0)),
        ],
        out_specs=pl.BlockSpec((_BM, _D), lambda i, k: (i, 0)),
        out_shape=jax.ShapeDtypeStruct((_N, _D), jnp.float32),
    )(dist_matrix, msg)
    return out


# speedup vs baseline: 1.1643x; 1.1643x over previous
import jax
import jax.numpy as jnp
from jax.experimental import pallas as pl

_N = 8192
_D = 64
_CUTOFF = 0.5
_MU = 1.0
_SIGMA = 0.5
_INV2SIG2 = 1.0 / (2.0 * _SIGMA * _SIGMA)

_BM = 256


def _msg_kernel(z_ref, w_ref, b_ref, out_ref):
    out_ref[...] = (
        jnp.dot(z_ref[...], w_ref[...], preferred_element_type=jnp.float32)
        + b_ref[...]
    ).astype(jnp.bfloat16)


def _interact_kernel(dist_ref, msg_ref, out_ref):
    r = dist_ref[...]
    u = 1.0 / r
    expo = -_INV2SIG2 * (u - 1.0 / _MU) ** 2
    w = jnp.where(r < _CUTOFF, jnp.exp(expo), 0.0).astype(jnp.bfloat16)
    out_ref[...] = jnp.dot(w, msg_ref[...], preferred_element_type=jnp.float32)


def kernel(z, dist_matrix, W, B):
    msg = pl.pallas_call(
        _msg_kernel,
        out_shape=jax.ShapeDtypeStruct((_N, _D), jnp.bfloat16),
    )(z, W, B.reshape(1, _D))

    out = pl.pallas_call(
        _interact_kernel,
        grid=(_N // _BM,),
        in_specs=[
            pl.BlockSpec((_BM, _N), lambda i: (i, 0)),
            pl.BlockSpec((_N, _D), lambda i: (0, 0)),
        ],
        out_specs=pl.BlockSpec((_BM, _D), lambda i: (i, 0)),
        out_shape=jax.ShapeDtypeStruct((_N, _D), jnp.float32),
    )(dist_matrix, msg)
    return out


# speedup vs baseline: 1.2029x; 1.0332x over previous
import jax
import jax.numpy as jnp
from jax.experimental import pallas as pl
from jax.experimental.pallas import tpu as pltpu

_N = 8192
_D = 64
_CUTOFF = 0.5
_MU = 1.0
_SIGMA = 0.5
_C2 = -1.4426950408889634 / (2.0 * _SIGMA * _SIGMA)

_BM = 256


def _msg_kernel(z_ref, w_ref, b_ref, out_ref):
    out_ref[...] = (
        jnp.dot(z_ref[...], w_ref[...], preferred_element_type=jnp.float32)
        + b_ref[...]
    ).astype(jnp.bfloat16)


def _interact_kernel(dist_ref, msg_ref, out_ref):
    r = dist_ref[...]
    u = 1.0 / r
    t = u - 1.0 / _MU
    w = jnp.where(r < _CUTOFF, jnp.exp2(_C2 * (t * t)), 0.0).astype(jnp.bfloat16)
    out_ref[...] = jnp.dot(w, msg_ref[...], preferred_element_type=jnp.float32)


def kernel(z, dist_matrix, W, B):
    msg = pl.pallas_call(
        _msg_kernel,
        out_shape=jax.ShapeDtypeStruct((_N, _D), jnp.bfloat16),
    )(z, W, B.reshape(1, _D))

    out = pl.pallas_call(
        _interact_kernel,
        grid=(_N // _BM,),
        in_specs=[
            pl.BlockSpec((_BM, _N), lambda i: (i, 0)),
            pl.BlockSpec((_N, _D), lambda i: (0, 0)),
        ],
        out_specs=pl.BlockSpec((_BM, _D), lambda i: (i, 0)),
        out_shape=jax.ShapeDtypeStruct((_N, _D), jnp.float32),
        compiler_params=pltpu.CompilerParams(
            dimension_semantics=("parallel",),
        ),
    )(dist_matrix, msg)
    return out
